# baseline (speedup 1.0000x reference)
import functools

import jax
import jax.numpy as jnp
from jax import lax
from jax.experimental import pallas as pl
from jax.experimental.pallas import tpu as pltpu
from jax.experimental.pallas import tpu_sc as plsc

N = 10000
F = 128
H = 128
B = 256
E = 160000
C = 86
L = 3

NP = N + 64
TILE = 64


_CH = 128
_NCH = 80
_EPW = _NCH * _CH
_EPAD = 16 * _EPW
_RPT = 632


@functools.cache
def _build_sc_segment_sum():
    @functools.partial(
        pl.kernel,
        out_type=jax.ShapeDtypeStruct((2 * N, H), jnp.float32),
        mesh=plsc.VectorSubcoreMesh(core_axis_name="c", subcore_axis_name="s"),
        scratch_types=[
            pltpu.VMEM((_CH,), jnp.int32),
            pltpu.VMEM((_CH,), jnp.int32),
            pltpu.VMEM((_CH, H), jnp.float32),
            pltpu.VMEM_SHARED((N, H), jnp.float32),
            pltpu.SemaphoreType.DMA,
        ],
    )
    def sc_seg_sum(h_hbm, src_hbm, dst_hbm, zeros_hbm, out_hbm,
                   idx_s, idx_d, rows, agg_sh, sem):
        c = lax.axis_index("c")
        s = lax.axis_index("s")
        w = c * 16 + s
        last = N - 15 * _RPT

        @pl.when(s < 15)
        def _():
            pltpu.sync_copy(zeros_hbm, agg_sh.at[pl.ds(s * _RPT, _RPT)])

        @pl.when(s == 15)
        def _():
            pltpu.sync_copy(zeros_hbm.at[pl.ds(0, last)],
                            agg_sh.at[pl.ds(15 * _RPT, last)])

        plsc.subcore_barrier()

        base = w * _EPW

        def chunk(i, carry):
            off = base + i * _CH
            pltpu.sync_copy(src_hbm.at[pl.ds(off, _CH)], idx_s)
            pltpu.sync_copy(dst_hbm.at[pl.ds(off, _CH)], idx_d)
            pltpu.async_copy(h_hbm.at[idx_s], rows, sem).wait()
            pltpu.sync_copy(rows, agg_sh.at[idx_d], add=True)
            return carry

        lax.fori_loop(0, _NCH, chunk, 0)

        plsc.subcore_barrier()

        @pl.when(s < 15)
        def _():
            pltpu.sync_copy(agg_sh.at[pl.ds(s * _RPT, _RPT)],
                            out_hbm.at[pl.ds(c * N + s * _RPT, _RPT)])

        @pl.when(s == 15)
        def _():
            pltpu.sync_copy(agg_sh.at[pl.ds(15 * _RPT, last)],
                            out_hbm.at[pl.ds(c * N + 15 * _RPT, last)])

    return sc_seg_sum


def _sc_segment_sum(h_cat, src, dst, zeros_blk):
    return _build_sc_segment_sum()(h_cat, src, dst, zeros_blk)


def _proj_body(x_ref, w_ref, b_ref, o_ref):
    o_ref[...] = jnp.dot(x_ref[...], w_ref[...],
                         preferred_element_type=jnp.float32) + b_ref[...]


def _input_proj(x_cat, W_in, b_in):
    blk = 2000 if (2 * N) % 2000 == 0 else 2 * N
    return pl.pallas_call(
        _proj_body,
        grid=(2 * N // blk,),
        in_specs=[
            pl.BlockSpec((blk, F), lambda i: (i, 0)),
            pl.BlockSpec((F, H), lambda i: (0, 0)),
            pl.BlockSpec((1, H), lambda i: (0, 0)),
        ],
        out_specs=pl.BlockSpec((blk, H), lambda i: (i, 0)),
        out_shape=jax.ShapeDtypeStruct((2 * N, H), jnp.float32),
    )(x_cat, W_in, b_in.reshape(1, H))


def _layer_body(agg_ref, h_ref, w_ref, b_ref, g_ref, be_ref, o_ref):
    agg = agg_ref[0]
    z = jnp.dot(agg, w_ref[...], preferred_element_type=jnp.float32) + b_ref[...]
    mu = jnp.mean(z, axis=0, keepdims=True)
    var = jnp.mean((z - mu) ** 2, axis=0, keepdims=True)
    hn = g_ref[...] * (z - mu) / jnp.sqrt(var + 1e-5) + be_ref[...]
    o_ref[...] = h_ref[...] + jnp.maximum(hn, 0.0)


def _layer_update(agg, h_cat, Wc_i, bc_i, gamma_i, beta_i):
    return pl.pallas_call(
        _layer_body,
        grid=(2,),
        in_specs=[
            pl.BlockSpec((1, N, H), lambda g: (g, 0, 0)),
            pl.BlockSpec((N, H), lambda g: (g, 0)),
            pl.BlockSpec((H, H), lambda g: (0, 0)),
            pl.BlockSpec((1, H), lambda g: (0, 0)),
            pl.BlockSpec((1, H), lambda g: (0, 0)),
            pl.BlockSpec((1, H), lambda g: (0, 0)),
        ],
        out_specs=pl.BlockSpec((N, H), lambda g: (g, 0)),
        out_shape=jax.ShapeDtypeStruct((2 * N, H), jnp.float32),
    )(agg.reshape(2, N, H), h_cat, Wc_i, bc_i.reshape(1, H),
      gamma_i.reshape(1, H), beta_i.reshape(1, H))


def _uv_body(h_ref, a_ref, b_ref, o_ref):
    o_ref[0] = jnp.dot(h_ref[0], a_ref[0],
                       preferred_element_type=jnp.float32) + b_ref[0]


def _uv_proj(hp12, A1s, ab1s):
    return pl.pallas_call(
        _uv_body,
        grid=(2,),
        in_specs=[
            pl.BlockSpec((1, NP, H), lambda g: (g, 0, 0)),
            pl.BlockSpec((1, H, H), lambda g: (g, 0, 0)),
            pl.BlockSpec((1, 1, H), lambda g: (g, 0, 0)),
        ],
        out_specs=pl.BlockSpec((1, NP, H), lambda g: (g, 0, 0)),
        out_shape=jax.ShapeDtypeStruct((2, NP, H), jnp.float32),
    )(hp12, A1s, ab1s)


def _attn_body(s1_ref, c1_ref, s2_ref, c2_ref,
               u_ref, v_ref, h1_ref, h2_ref, a2_ref, ab2_ref,
               g1_ref, g2_ref, att1_buf, att2_buf):
    b = pl.program_id(0)
    s1 = s1_ref[b]
    n1 = c1_ref[b]
    s2 = s2_ref[b]
    n2 = c2_ref[b]
    nr1 = (n1 + TILE - 1) // TILE
    nc2 = (n2 + TILE - 1) // TILE

    a2 = a2_ref[...]
    ab2 = ab2_ref[0, 0]
    smax = jnp.sum(jnp.abs(a2)) + jnp.abs(ab2)

    def zbody(ct, carry):
        att2_buf[pl.ds(ct * TILE, TILE), :] = jnp.zeros((TILE, 1), jnp.float32)
        return carry
    lax.fori_loop(0, nc2, zbody, 0)

    def rbody(rt, denom):
        u_t = u_ref[pl.ds(s1 + rt * TILE, TILE), :]
        rrem = n1 - rt * TILE

        def cbody(ct, carry):
            att1_acc, dn = carry
            v_t = v_ref[pl.ds(s2 + ct * TILE, TILE), :]
            crem = n2 - ct * TILE
            t3 = jnp.tanh(u_t[:, None, :] + v_t[None, :, :])
            S = jnp.sum(t3 * a2[None, :, :], axis=-1) + ab2
            rmask = lax.broadcasted_iota(jnp.int32, (TILE, TILE), 0) < rrem
            cmask = lax.broadcasted_iota(jnp.int32, (TILE, TILE), 1) < crem
            e = jnp.where(rmask & cmask, jnp.exp(S - smax), 0.0)
            att1_acc = att1_acc + jnp.sum(e, axis=1, keepdims=True)
            col = jnp.sum(e.T, axis=1, keepdims=True)
            att2_buf[pl.ds(ct * TILE, TILE), :] = (
                att2_buf[pl.ds(ct * TILE, TILE), :] + col)
            return att1_acc, dn + jnp.sum(e)

        att1_acc, denom = lax.fori_loop(
            0, nc2, cbody, (jnp.zeros((TILE, 1), jnp.float32), denom))
        att1_buf[pl.ds(rt * TILE, TILE), :] = att1_acc
        return denom

    denom = lax.fori_loop(0, nr1, rbody, jnp.float32(0.0))
    dsafe = jnp.where(denom > 0.0, denom, 1.0)

    def g1body(rt, acc):
        w = att1_buf[pl.ds(rt * TILE, TILE), :] / dsafe
        h_t = h1_ref[pl.ds(s1 + rt * TILE, TILE), :]
        return acc + jnp.sum(w * h_t, axis=0, keepdims=True)

    g1_ref[0] = lax.fori_loop(0, nr1, g1body, jnp.zeros((1, H), jnp.float32))

    def g2body(ct, acc):
        w = att2_buf[pl.ds(ct * TILE, TILE), :] / dsafe
        h_t = h2_ref[pl.ds(s2 + ct * TILE, TILE), :]
        return acc + jnp.sum(w * h_t, axis=0, keepdims=True)

    g2_ref[0] = lax.fori_loop(0, nc2, g2body, jnp.zeros((1, H), jnp.float32))


def _attention(starts1, counts1, starts2, counts2, u, v, h1p, h2p, a2t, ab2m):
    grid_spec = pltpu.PrefetchScalarGridSpec(
        num_scalar_prefetch=4,
        grid=(B,),
        in_specs=[
            pl.BlockSpec((NP, H), lambda b, *_: (0, 0)),
            pl.BlockSpec((NP, H), lambda b, *_: (0, 0)),
            pl.BlockSpec((NP, H), lambda b, *_: (0, 0)),
            pl.BlockSpec((NP, H), lambda b, *_: (0, 0)),
            pl.BlockSpec((1, H), lambda b, *_: (0, 0)),
            pl.BlockSpec((1, 1), lambda b, *_: (0, 0)),
        ],
        out_specs=[
            pl.BlockSpec((1, 1, H), lambda b, *_: (b, 0, 0)),
            pl.BlockSpec((1, 1, H), lambda b, *_: (b, 0, 0)),
        ],
        scratch_shapes=[
            pltpu.VMEM((NP, 1), jnp.float32),
            pltpu.VMEM((NP, 1), jnp.float32),
        ],
    )
    g1, g2 = pl.pallas_call(
        _attn_body,
        grid_spec=grid_spec,
        out_shape=[
            jax.ShapeDtypeStruct((B, 1, H), jnp.float32),
            jax.ShapeDtypeStruct((B, 1, H), jnp.float32),
        ],
    )(starts1, counts1, starts2, counts2, u, v, h1p, h2p, a2t, ab2m)
    return g1.reshape(B, H), g2.reshape(B, H)


def _mlp_body(g1_ref, g2_ref, c1a_ref, c1b_ref, cb1_ref, c2_ref, cb2_ref, o_ref):
    hp = jnp.dot(g1_ref[...], c1a_ref[...], preferred_element_type=jnp.float32)
    hp = hp + jnp.dot(g2_ref[...], c1b_ref[...], preferred_element_type=jnp.float32)
    hp = jnp.maximum(hp + cb1_ref[...], 0.0)
    o_ref[...] = jnp.dot(hp, c2_ref[...],
                         preferred_element_type=jnp.float32) + cb2_ref[...]


def _final_mlp(g1, g2, C1, cb1, C2, cb2):
    return pl.pallas_call(
        _mlp_body,
        out_shape=jax.ShapeDtypeStruct((B, C), jnp.float32),
    )(g1, g2, C1[:H], C1[H:], cb1.reshape(1, H), C2, cb2.reshape(1, C))


def kernel(x1, x2, edge_index1, edge_index2, batch1, batch2, W_in, b_in, Wc, bc,
           gamma, beta, A1, ab1, A2, ab2, C1, cb1, C2, cb2):
    x_cat = jnp.concatenate([x1, x2], axis=0)
    pad = _EPAD - E
    spad = jnp.full((pad,), 2 * N, jnp.int32)
    dpad = jnp.arange(pad, dtype=jnp.int32) % N
    src1 = jnp.concatenate([edge_index1[0].astype(jnp.int32), spad])
    src2 = jnp.concatenate([edge_index2[0].astype(jnp.int32) + N, spad])
    dst1 = jnp.concatenate([edge_index1[1].astype(jnp.int32), dpad])
    dst2 = jnp.concatenate([edge_index2[1].astype(jnp.int32), dpad])
    src = jnp.stack([src1, src2]).reshape(-1)
    dst = jnp.stack([dst1, dst2]).reshape(-1)
    zeros_blk = jnp.zeros((_RPT, H), jnp.float32)
    zrow = jnp.zeros((8, H), jnp.float32)

    h_cat = _input_proj(x_cat, W_in, b_in)
    for i in range(L):
        agg = _sc_segment_sum(jnp.concatenate([h_cat, zrow]), src, dst,
                              zeros_blk)
        h_cat = _layer_update(agg, h_cat, Wc[i], bc[i], gamma[i], beta[i])

    h1 = h_cat[:N]
    h2 = h_cat[N:]
    hp1 = jnp.zeros((NP, H), jnp.float32).at[:N].set(h1)
    hp2 = jnp.zeros((NP, H), jnp.float32).at[:N].set(h2)
    hp12 = jnp.stack([hp1, hp2])
    A1s = A1.reshape(2, H, H)
    ab1s = jnp.stack([ab1.reshape(1, H), jnp.zeros((1, H), jnp.float32)])
    uv = _uv_proj(hp12, A1s, ab1s)
    u, v = uv[0], uv[1]

    ar = jnp.arange(B, dtype=batch1.dtype)
    starts1 = jnp.searchsorted(batch1, ar, side="left").astype(jnp.int32)
    counts1 = jnp.searchsorted(batch1, ar, side="right").astype(jnp.int32) - starts1
    starts2 = jnp.searchsorted(batch2, ar, side="left").astype(jnp.int32)
    counts2 = jnp.searchsorted(batch2, ar, side="right").astype(jnp.int32) - starts2

    g1, g2 = _attention(starts1, counts1, starts2, counts2, u, v, hp1, hp2,
                        A2.reshape(1, H), ab2.reshape(1, 1))
    return _final_mlp(g1, g2, C1, cb1, C2, cb2)


# speedup vs baseline: 1.5454x; 1.5454x over previous
import functools

import jax
import jax.numpy as jnp
from jax import lax
from jax.experimental import pallas as pl
from jax.experimental.pallas import tpu as pltpu
from jax.experimental.pallas import tpu_sc as plsc

N = 10000
F = 128
H = 128
B = 256
E = 160000
C = 86
L = 3

NP = N + 64
TILE = 64


_CH = 128
_NCH = 80
_EPW = _NCH * _CH
_EPAD = 16 * _EPW
_RPT = 632


@functools.cache
def _build_sc_segment_sum():
    @functools.partial(
        pl.kernel,
        out_type=jax.ShapeDtypeStruct((2 * N, H), jnp.float32),
        mesh=plsc.VectorSubcoreMesh(core_axis_name="c", subcore_axis_name="s"),
        scratch_types=[
            pltpu.VMEM((_CH,), jnp.int32),
            pltpu.VMEM((_CH,), jnp.int32),
            pltpu.VMEM((_CH, H), jnp.float32),
            pltpu.VMEM_SHARED((N, H), jnp.float32),
            pltpu.SemaphoreType.DMA,
        ],
    )
    def sc_seg_sum(h_hbm, src_hbm, dst_hbm, zeros_hbm, out_hbm,
                   idx_s, idx_d, rows, agg_sh, sem):
        c = lax.axis_index("c")
        s = lax.axis_index("s")
        w = c * 16 + s
        last = N - 15 * _RPT

        @pl.when(s < 15)
        def _():
            pltpu.sync_copy(zeros_hbm, agg_sh.at[pl.ds(s * _RPT, _RPT)])

        @pl.when(s == 15)
        def _():
            pltpu.sync_copy(zeros_hbm.at[pl.ds(0, last)],
                            agg_sh.at[pl.ds(15 * _RPT, last)])

        plsc.subcore_barrier()

        base = w * _EPW

        def chunk(i, carry):
            off = base + i * _CH
            pltpu.sync_copy(src_hbm.at[pl.ds(off, _CH)], idx_s)
            pltpu.sync_copy(dst_hbm.at[pl.ds(off, _CH)], idx_d)
            pltpu.async_copy(h_hbm.at[idx_s], rows, sem).wait()
            pltpu.sync_copy(rows, agg_sh.at[idx_d], add=True)
            return carry

        lax.fori_loop(0, _NCH, chunk, 0)

        plsc.subcore_barrier()

        @pl.when(s < 15)
        def _():
            pltpu.sync_copy(agg_sh.at[pl.ds(s * _RPT, _RPT)],
                            out_hbm.at[pl.ds(c * N + s * _RPT, _RPT)])

        @pl.when(s == 15)
        def _():
            pltpu.sync_copy(agg_sh.at[pl.ds(15 * _RPT, last)],
                            out_hbm.at[pl.ds(c * N + 15 * _RPT, last)])

    return sc_seg_sum


def _sc_segment_sum(h_cat, src, dst, zeros_blk):
    return _build_sc_segment_sum()(h_cat, src, dst, zeros_blk)


def _proj_body(x_ref, w_ref, b_ref, o_ref):
    o_ref[...] = jnp.dot(x_ref[...], w_ref[...],
                         preferred_element_type=jnp.float32) + b_ref[...]


def _input_proj(x_cat, W_in, b_in):
    blk = 2000 if (2 * N) % 2000 == 0 else 2 * N
    return pl.pallas_call(
        _proj_body,
        grid=(2 * N // blk,),
        in_specs=[
            pl.BlockSpec((blk, F), lambda i: (i, 0)),
            pl.BlockSpec((F, H), lambda i: (0, 0)),
            pl.BlockSpec((1, H), lambda i: (0, 0)),
        ],
        out_specs=pl.BlockSpec((blk, H), lambda i: (i, 0)),
        out_shape=jax.ShapeDtypeStruct((2 * N, H), jnp.float32),
    )(x_cat, W_in, b_in.reshape(1, H))


def _layer_body(agg_ref, h_ref, w_ref, b_ref, g_ref, be_ref, o_ref):
    agg = agg_ref[0]
    z = jnp.dot(agg, w_ref[...], preferred_element_type=jnp.float32) + b_ref[...]
    mu = jnp.mean(z, axis=0, keepdims=True)
    var = jnp.mean((z - mu) ** 2, axis=0, keepdims=True)
    hn = g_ref[...] * (z - mu) / jnp.sqrt(var + 1e-5) + be_ref[...]
    o_ref[...] = h_ref[...] + jnp.maximum(hn, 0.0)


def _layer_update(agg, h_cat, Wc_i, bc_i, gamma_i, beta_i):
    return pl.pallas_call(
        _layer_body,
        grid=(2,),
        in_specs=[
            pl.BlockSpec((1, N, H), lambda g: (g, 0, 0)),
            pl.BlockSpec((N, H), lambda g: (g, 0)),
            pl.BlockSpec((H, H), lambda g: (0, 0)),
            pl.BlockSpec((1, H), lambda g: (0, 0)),
            pl.BlockSpec((1, H), lambda g: (0, 0)),
            pl.BlockSpec((1, H), lambda g: (0, 0)),
        ],
        out_specs=pl.BlockSpec((N, H), lambda g: (g, 0)),
        out_shape=jax.ShapeDtypeStruct((2 * N, H), jnp.float32),
    )(agg.reshape(2, N, H), h_cat, Wc_i, bc_i.reshape(1, H),
      gamma_i.reshape(1, H), beta_i.reshape(1, H))


def _uv_body(h_ref, a_ref, b_ref, o_ref):
    o_ref[0] = jnp.dot(h_ref[0], a_ref[0],
                       preferred_element_type=jnp.float32) + b_ref[0]


def _uv_proj(hp12, A1s, ab1s):
    return pl.pallas_call(
        _uv_body,
        grid=(2,),
        in_specs=[
            pl.BlockSpec((1, NP, H), lambda g: (g, 0, 0)),
            pl.BlockSpec((1, H, H), lambda g: (g, 0, 0)),
            pl.BlockSpec((1, 1, H), lambda g: (g, 0, 0)),
        ],
        out_specs=pl.BlockSpec((1, NP, H), lambda g: (g, 0, 0)),
        out_shape=jax.ShapeDtypeStruct((2, NP, H), jnp.float32),
    )(hp12, A1s, ab1s)


def _attn_body(s1_ref, c1_ref, s2_ref, c2_ref,
               u_ref, v_ref, h1_ref, h2_ref, a2_ref, ab2_ref,
               g1_ref, g2_ref, att1_buf, att2_buf):
    b = pl.program_id(0)
    s1 = s1_ref[b]
    n1 = c1_ref[b]
    s2 = s2_ref[b]
    n2 = c2_ref[b]
    nr1 = (n1 + TILE - 1) // TILE
    nc2 = (n2 + TILE - 1) // TILE

    a2 = a2_ref[...]
    ab2 = ab2_ref[0, 0]
    smax = jnp.sum(jnp.abs(a2)) + jnp.abs(ab2)

    def zbody(ct, carry):
        att2_buf[pl.ds(ct * TILE, TILE), :] = jnp.zeros((TILE, 1), jnp.float32)
        return carry
    lax.fori_loop(0, nc2, zbody, 0)

    def rbody(rt, denom):
        u_t = u_ref[pl.ds(s1 + rt * TILE, TILE), :]
        rrem = n1 - rt * TILE

        def cbody(ct, carry):
            att1_acc, dn = carry
            v_t = v_ref[pl.ds(s2 + ct * TILE, TILE), :]
            crem = n2 - ct * TILE
            t3 = jnp.tanh(u_t[:, None, :] + v_t[None, :, :])
            S = jnp.sum(t3 * a2[None, :, :], axis=-1) + ab2
            rmask = lax.broadcasted_iota(jnp.int32, (TILE, TILE), 0) < rrem
            cmask = lax.broadcasted_iota(jnp.int32, (TILE, TILE), 1) < crem
            e = jnp.where(rmask & cmask, jnp.exp(S - smax), 0.0)
            att1_acc = att1_acc + jnp.sum(e, axis=1, keepdims=True)
            col = jnp.sum(e.T, axis=1, keepdims=True)
            att2_buf[pl.ds(ct * TILE, TILE), :] = (
                att2_buf[pl.ds(ct * TILE, TILE), :] + col)
            return att1_acc, dn + jnp.sum(e)

        att1_acc, denom = lax.fori_loop(
            0, nc2, cbody, (jnp.zeros((TILE, 1), jnp.float32), denom))
        att1_buf[pl.ds(rt * TILE, TILE), :] = att1_acc
        return denom

    denom = lax.fori_loop(0, nr1, rbody, jnp.float32(0.0))
    dsafe = jnp.where(denom > 0.0, denom, 1.0)

    def g1body(rt, acc):
        w = att1_buf[pl.ds(rt * TILE, TILE), :] / dsafe
        h_t = h1_ref[pl.ds(s1 + rt * TILE, TILE), :]
        return acc + jnp.sum(w * h_t, axis=0, keepdims=True)

    g1_ref[0] = lax.fori_loop(0, nr1, g1body, jnp.zeros((1, H), jnp.float32))

    def g2body(ct, acc):
        w = att2_buf[pl.ds(ct * TILE, TILE), :] / dsafe
        h_t = h2_ref[pl.ds(s2 + ct * TILE, TILE), :]
        return acc + jnp.sum(w * h_t, axis=0, keepdims=True)

    g2_ref[0] = lax.fori_loop(0, nc2, g2body, jnp.zeros((1, H), jnp.float32))


def _attention(starts1, counts1, starts2, counts2, u, v, h1p, h2p, a2t, ab2m):
    grid_spec = pltpu.PrefetchScalarGridSpec(
        num_scalar_prefetch=4,
        grid=(B,),
        in_specs=[
            pl.BlockSpec((NP, H), lambda b, *_: (0, 0)),
            pl.BlockSpec((NP, H), lambda b, *_: (0, 0)),
            pl.BlockSpec((NP, H), lambda b, *_: (0, 0)),
            pl.BlockSpec((NP, H), lambda b, *_: (0, 0)),
            pl.BlockSpec((1, H), lambda b, *_: (0, 0)),
            pl.BlockSpec((1, 1), lambda b, *_: (0, 0)),
        ],
        out_specs=[
            pl.BlockSpec((1, 1, H), lambda b, *_: (b, 0, 0)),
            pl.BlockSpec((1, 1, H), lambda b, *_: (b, 0, 0)),
        ],
        scratch_shapes=[
            pltpu.VMEM((NP, 1), jnp.float32),
            pltpu.VMEM((NP, 1), jnp.float32),
        ],
    )
    g1, g2 = pl.pallas_call(
        _attn_body,
        grid_spec=grid_spec,
        out_shape=[
            jax.ShapeDtypeStruct((B, 1, H), jnp.float32),
            jax.ShapeDtypeStruct((B, 1, H), jnp.float32),
        ],
    )(starts1, counts1, starts2, counts2, u, v, h1p, h2p, a2t, ab2m)
    return g1.reshape(B, H), g2.reshape(B, H)


def _mlp_body(g1_ref, g2_ref, c1a_ref, c1b_ref, cb1_ref, c2_ref, cb2_ref, o_ref):
    hp = jnp.dot(g1_ref[...], c1a_ref[...], preferred_element_type=jnp.float32)
    hp = hp + jnp.dot(g2_ref[...], c1b_ref[...], preferred_element_type=jnp.float32)
    hp = jnp.maximum(hp + cb1_ref[...], 0.0)
    o_ref[...] = jnp.dot(hp, c2_ref[...],
                         preferred_element_type=jnp.float32) + cb2_ref[...]


def _final_mlp(g1, g2, C1, cb1, C2, cb2):
    return pl.pallas_call(
        _mlp_body,
        out_shape=jax.ShapeDtypeStruct((B, C), jnp.float32),
    )(g1, g2, C1[:H], C1[H:], cb1.reshape(1, H), C2, cb2.reshape(1, C))


def kernel(x1, x2, edge_index1, edge_index2, batch1, batch2, W_in, b_in, Wc, bc,
           gamma, beta, A1, ab1, A2, ab2, C1, cb1, C2, cb2):
    x_cat = jnp.concatenate([x1, x2], axis=0)
    pad = _EPAD - E
    spad = 2 * N + jnp.arange(pad, dtype=jnp.int32) % 128
    dpad = jnp.arange(pad, dtype=jnp.int32) % N
    src1 = jnp.concatenate([edge_index1[0].astype(jnp.int32), spad])
    src2 = jnp.concatenate([edge_index2[0].astype(jnp.int32) + N, spad])
    dst1 = jnp.concatenate([edge_index1[1].astype(jnp.int32), dpad])
    dst2 = jnp.concatenate([edge_index2[1].astype(jnp.int32), dpad])
    src = jnp.stack([src1, src2]).reshape(-1)
    dst = jnp.stack([dst1, dst2]).reshape(-1)
    zeros_blk = jnp.zeros((_RPT, H), jnp.float32)
    zrow = jnp.zeros((128, H), jnp.float32)

    h_cat = _input_proj(x_cat, W_in, b_in)
    for i in range(L):
        agg = _sc_segment_sum(jnp.concatenate([h_cat, zrow]), src, dst,
                              zeros_blk)
        h_cat = _layer_update(agg, h_cat, Wc[i], bc[i], gamma[i], beta[i])

    h1 = h_cat[:N]
    h2 = h_cat[N:]
    hp1 = jnp.zeros((NP, H), jnp.float32).at[:N].set(h1)
    hp2 = jnp.zeros((NP, H), jnp.float32).at[:N].set(h2)
    hp12 = jnp.stack([hp1, hp2])
    A1s = A1.reshape(2, H, H)
    ab1s = jnp.stack([ab1.reshape(1, H), jnp.zeros((1, H), jnp.float32)])
    uv = _uv_proj(hp12, A1s, ab1s)
    u, v = uv[0], uv[1]

    ar = jnp.arange(B, dtype=batch1.dtype)
    starts1 = jnp.searchsorted(batch1, ar, side="left").astype(jnp.int32)
    counts1 = jnp.searchsorted(batch1, ar, side="right").astype(jnp.int32) - starts1
    starts2 = jnp.searchsorted(batch2, ar, side="left").astype(jnp.int32)
    counts2 = jnp.searchsorted(batch2, ar, side="right").astype(jnp.int32) - starts2

    g1, g2 = _attention(starts1, counts1, starts2, counts2, u, v, hp1, hp2,
                        A2.reshape(1, H), ab2.reshape(1, 1))
    return _final_mlp(g1, g2, C1, cb1, C2, cb2)


# speedup vs baseline: 2.3257x; 1.5049x over previous
import functools

import jax
import jax.numpy as jnp
from jax import lax
from jax.experimental import pallas as pl
from jax.experimental.pallas import tpu as pltpu
from jax.experimental.pallas import tpu_sc as plsc

N = 10000
F = 128
H = 128
B = 256
E = 160000
C = 86
L = 3

NP = N + 64
TILE = 64


_CH = 128
_NCH = 80
_EPW = _NCH * _CH
_EPAD = 16 * _EPW
_RPT = 632


@functools.cache
def _build_sc_segment_sum():
    @functools.partial(
        pl.kernel,
        out_type=jax.ShapeDtypeStruct((2 * N, H), jnp.float32),
        mesh=plsc.VectorSubcoreMesh(core_axis_name="c", subcore_axis_name="s"),
        scratch_types=[
            pltpu.VMEM((_NCH // 2, _CH), jnp.int32),
            pltpu.VMEM((_NCH // 2, _CH), jnp.int32),
            pltpu.VMEM((_CH, H), jnp.float32),
            pltpu.VMEM((_CH, H), jnp.float32),
            pltpu.VMEM_SHARED((N, H), jnp.float32),
            pltpu.SemaphoreType.DMA,
            pltpu.SemaphoreType.DMA,
        ],
    )
    def sc_seg_sum(h_hbm, src_hbm, dst_hbm, zeros_hbm, out_hbm,
                   idx_s, idx_d, rows0, rows1, agg_sh, sem0, sem1):
        c = lax.axis_index("c")
        s = lax.axis_index("s")
        w = c * 16 + s
        last = N - 15 * _RPT

        @pl.when(s < 15)
        def _():
            pltpu.sync_copy(zeros_hbm, agg_sh.at[pl.ds(s * _RPT, _RPT)])

        @pl.when(s == 15)
        def _():
            pltpu.sync_copy(zeros_hbm.at[pl.ds(0, last)],
                            agg_sh.at[pl.ds(15 * _RPT, last)])

        plsc.subcore_barrier()
        half = _NCH // 2
        for p in range(2):
            pltpu.sync_copy(src_hbm.at[w, p], idx_s)
            pltpu.sync_copy(dst_hbm.at[w, p], idx_d)

            pltpu.make_async_copy(h_hbm.at[idx_s.at[0]], rows0, sem0).start()

            def body(j, carry):
                i0 = 2 * j
                pltpu.make_async_copy(h_hbm.at[idx_s.at[i0 + 1]], rows1,
                                      sem1).start()
                pltpu.make_async_copy(h_hbm.at[idx_s.at[i0]], rows0,
                                      sem0).wait()
                pltpu.sync_copy(rows0, agg_sh.at[idx_d.at[i0]], add=True)

                @pl.when(i0 + 2 < half)
                def _():
                    pltpu.make_async_copy(h_hbm.at[idx_s.at[i0 + 2]], rows0,
                                          sem0).start()

                pltpu.make_async_copy(h_hbm.at[idx_s.at[i0 + 1]], rows1,
                                      sem1).wait()
                pltpu.sync_copy(rows1, agg_sh.at[idx_d.at[i0 + 1]], add=True)
                return carry

            lax.fori_loop(0, half // 2, body, 0)

        plsc.subcore_barrier()

        @pl.when(s < 15)
        def _():
            pltpu.sync_copy(agg_sh.at[pl.ds(s * _RPT, _RPT)],
                            out_hbm.at[pl.ds(c * N + s * _RPT, _RPT)])

        @pl.when(s == 15)
        def _():
            pltpu.sync_copy(agg_sh.at[pl.ds(15 * _RPT, last)],
                            out_hbm.at[pl.ds(c * N + 15 * _RPT, last)])

    return sc_seg_sum


def _sc_segment_sum(h_cat, src, dst, zeros_blk):
    return _build_sc_segment_sum()(h_cat, src, dst, zeros_blk)


def _proj_body(x_ref, w_ref, b_ref, o_ref):
    o_ref[...] = jnp.dot(x_ref[...], w_ref[...],
                         preferred_element_type=jnp.float32) + b_ref[...]


def _input_proj(x_cat, W_in, b_in):
    blk = 2000 if (2 * N) % 2000 == 0 else 2 * N
    return pl.pallas_call(
        _proj_body,
        grid=(2 * N // blk,),
        in_specs=[
            pl.BlockSpec((blk, F), lambda i: (i, 0)),
            pl.BlockSpec((F, H), lambda i: (0, 0)),
            pl.BlockSpec((1, H), lambda i: (0, 0)),
        ],
        out_specs=pl.BlockSpec((blk, H), lambda i: (i, 0)),
        out_shape=jax.ShapeDtypeStruct((2 * N, H), jnp.float32),
    )(x_cat, W_in, b_in.reshape(1, H))


def _layer_body(agg_ref, h_ref, w_ref, b_ref, g_ref, be_ref, o_ref):
    agg = agg_ref[0]
    z = jnp.dot(agg, w_ref[...], preferred_element_type=jnp.float32) + b_ref[...]
    mu = jnp.mean(z, axis=0, keepdims=True)
    var = jnp.mean((z - mu) ** 2, axis=0, keepdims=True)
    hn = g_ref[...] * (z - mu) / jnp.sqrt(var + 1e-5) + be_ref[...]
    o_ref[...] = h_ref[...] + jnp.maximum(hn, 0.0)


def _layer_update(agg, h_cat, Wc_i, bc_i, gamma_i, beta_i):
    return pl.pallas_call(
        _layer_body,
        grid=(2,),
        in_specs=[
            pl.BlockSpec((1, N, H), lambda g: (g, 0, 0)),
            pl.BlockSpec((N, H), lambda g: (g, 0)),
            pl.BlockSpec((H, H), lambda g: (0, 0)),
            pl.BlockSpec((1, H), lambda g: (0, 0)),
            pl.BlockSpec((1, H), lambda g: (0, 0)),
            pl.BlockSpec((1, H), lambda g: (0, 0)),
        ],
        out_specs=pl.BlockSpec((N, H), lambda g: (g, 0)),
        out_shape=jax.ShapeDtypeStruct((2 * N, H), jnp.float32),
    )(agg.reshape(2, N, H), h_cat, Wc_i, bc_i.reshape(1, H),
      gamma_i.reshape(1, H), beta_i.reshape(1, H))


def _uv_body(h_ref, a_ref, b_ref, o_ref):
    o_ref[0] = jnp.dot(h_ref[0], a_ref[0],
                       preferred_element_type=jnp.float32) + b_ref[0]


def _uv_proj(hp12, A1s, ab1s):
    return pl.pallas_call(
        _uv_body,
        grid=(2,),
        in_specs=[
            pl.BlockSpec((1, NP, H), lambda g: (g, 0, 0)),
            pl.BlockSpec((1, H, H), lambda g: (g, 0, 0)),
            pl.BlockSpec((1, 1, H), lambda g: (g, 0, 0)),
        ],
        out_specs=pl.BlockSpec((1, NP, H), lambda g: (g, 0, 0)),
        out_shape=jax.ShapeDtypeStruct((2, NP, H), jnp.float32),
    )(hp12, A1s, ab1s)


def _attn_body(s1_ref, c1_ref, s2_ref, c2_ref,
               u_ref, v_ref, h1_ref, h2_ref, a2_ref, ab2_ref,
               g1_ref, g2_ref, att1_buf, att2_buf):
    b = pl.program_id(0)
    s1 = s1_ref[b]
    n1 = c1_ref[b]
    s2 = s2_ref[b]
    n2 = c2_ref[b]
    nr1 = (n1 + TILE - 1) // TILE
    nc2 = (n2 + TILE - 1) // TILE

    a2 = a2_ref[...]
    ab2 = ab2_ref[0, 0]
    smax = jnp.sum(jnp.abs(a2)) + jnp.abs(ab2)

    def zbody(ct, carry):
        att2_buf[pl.ds(ct * TILE, TILE), :] = jnp.zeros((TILE, 1), jnp.float32)
        return carry
    lax.fori_loop(0, nc2, zbody, 0)

    def rbody(rt, denom):
        u_t = u_ref[pl.ds(s1 + rt * TILE, TILE), :]
        rrem = n1 - rt * TILE

        def cbody(ct, carry):
            att1_acc, dn = carry
            v_t = v_ref[pl.ds(s2 + ct * TILE, TILE), :]
            crem = n2 - ct * TILE
            t3 = jnp.tanh(u_t[:, None, :] + v_t[None, :, :])
            S = jnp.sum(t3 * a2[None, :, :], axis=-1) + ab2
            rmask = lax.broadcasted_iota(jnp.int32, (TILE, TILE), 0) < rrem
            cmask = lax.broadcasted_iota(jnp.int32, (TILE, TILE), 1) < crem
            e = jnp.where(rmask & cmask, jnp.exp(S - smax), 0.0)
            att1_acc = att1_acc + jnp.sum(e, axis=1, keepdims=True)
            col = jnp.sum(e.T, axis=1, keepdims=True)
            att2_buf[pl.ds(ct * TILE, TILE), :] = (
                att2_buf[pl.ds(ct * TILE, TILE), :] + col)
            return att1_acc, dn + jnp.sum(e)

        att1_acc, denom = lax.fori_loop(
            0, nc2, cbody, (jnp.zeros((TILE, 1), jnp.float32), denom))
        att1_buf[pl.ds(rt * TILE, TILE), :] = att1_acc
        return denom

    denom = lax.fori_loop(0, nr1, rbody, jnp.float32(0.0))
    dsafe = jnp.where(denom > 0.0, denom, 1.0)

    def g1body(rt, acc):
        w = att1_buf[pl.ds(rt * TILE, TILE), :] / dsafe
        h_t = h1_ref[pl.ds(s1 + rt * TILE, TILE), :]
        return acc + jnp.sum(w * h_t, axis=0, keepdims=True)

    g1_ref[0] = lax.fori_loop(0, nr1, g1body, jnp.zeros((1, H), jnp.float32))

    def g2body(ct, acc):
        w = att2_buf[pl.ds(ct * TILE, TILE), :] / dsafe
        h_t = h2_ref[pl.ds(s2 + ct * TILE, TILE), :]
        return acc + jnp.sum(w * h_t, axis=0, keepdims=True)

    g2_ref[0] = lax.fori_loop(0, nc2, g2body, jnp.zeros((1, H), jnp.float32))


def _attention(starts1, counts1, starts2, counts2, u, v, h1p, h2p, a2t, ab2m):
    grid_spec = pltpu.PrefetchScalarGridSpec(
        num_scalar_prefetch=4,
        grid=(B,),
        in_specs=[
            pl.BlockSpec((NP, H), lambda b, *_: (0, 0)),
            pl.BlockSpec((NP, H), lambda b, *_: (0, 0)),
            pl.BlockSpec((NP, H), lambda b, *_: (0, 0)),
            pl.BlockSpec((NP, H), lambda b, *_: (0, 0)),
            pl.BlockSpec((1, H), lambda b, *_: (0, 0)),
            pl.BlockSpec((1, 1), lambda b, *_: (0, 0)),
        ],
        out_specs=[
            pl.BlockSpec((1, 1, H), lambda b, *_: (b, 0, 0)),
            pl.BlockSpec((1, 1, H), lambda b, *_: (b, 0, 0)),
        ],
        scratch_shapes=[
            pltpu.VMEM((NP, 1), jnp.float32),
            pltpu.VMEM((NP, 1), jnp.float32),
        ],
    )
    g1, g2 = pl.pallas_call(
        _attn_body,
        grid_spec=grid_spec,
        out_shape=[
            jax.ShapeDtypeStruct((B, 1, H), jnp.float32),
            jax.ShapeDtypeStruct((B, 1, H), jnp.float32),
        ],
    )(starts1, counts1, starts2, counts2, u, v, h1p, h2p, a2t, ab2m)
    return g1.reshape(B, H), g2.reshape(B, H)


def _mlp_body(g1_ref, g2_ref, c1a_ref, c1b_ref, cb1_ref, c2_ref, cb2_ref, o_ref):
    hp = jnp.dot(g1_ref[...], c1a_ref[...], preferred_element_type=jnp.float32)
    hp = hp + jnp.dot(g2_ref[...], c1b_ref[...], preferred_element_type=jnp.float32)
    hp = jnp.maximum(hp + cb1_ref[...], 0.0)
    o_ref[...] = jnp.dot(hp, c2_ref[...],
                         preferred_element_type=jnp.float32) + cb2_ref[...]


def _final_mlp(g1, g2, C1, cb1, C2, cb2):
    return pl.pallas_call(
        _mlp_body,
        out_shape=jax.ShapeDtypeStruct((B, C), jnp.float32),
    )(g1, g2, C1[:H], C1[H:], cb1.reshape(1, H), C2, cb2.reshape(1, C))


def kernel(x1, x2, edge_index1, edge_index2, batch1, batch2, W_in, b_in, Wc, bc,
           gamma, beta, A1, ab1, A2, ab2, C1, cb1, C2, cb2):
    x_cat = jnp.concatenate([x1, x2], axis=0)
    pad = _EPAD - E
    spad = 2 * N + jnp.arange(pad, dtype=jnp.int32) % 128
    dpad = jnp.arange(pad, dtype=jnp.int32) % N
    src1 = jnp.concatenate([edge_index1[0].astype(jnp.int32), spad])
    src2 = jnp.concatenate([edge_index2[0].astype(jnp.int32) + N, spad])
    dst1 = jnp.concatenate([edge_index1[1].astype(jnp.int32), dpad])
    dst2 = jnp.concatenate([edge_index2[1].astype(jnp.int32), dpad])
    src = jnp.stack([src1, src2]).reshape(32, 2, _NCH // 2, _CH)
    dst = jnp.stack([dst1, dst2]).reshape(32, 2, _NCH // 2, _CH)
    zeros_blk = jnp.zeros((_RPT, H), jnp.float32)
    zrow = jnp.zeros((128, H), jnp.float32)

    h_cat = _input_proj(x_cat, W_in, b_in)
    for i in range(L):
        agg = _sc_segment_sum(jnp.concatenate([h_cat, zrow]), src, dst,
                              zeros_blk)
        h_cat = _layer_update(agg, h_cat, Wc[i], bc[i], gamma[i], beta[i])

    h1 = h_cat[:N]
    h2 = h_cat[N:]
    hp1 = jnp.zeros((NP, H), jnp.float32).at[:N].set(h1)
    hp2 = jnp.zeros((NP, H), jnp.float32).at[:N].set(h2)
    hp12 = jnp.stack([hp1, hp2])
    A1s = A1.reshape(2, H, H)
    ab1s = jnp.stack([ab1.reshape(1, H), jnp.zeros((1, H), jnp.float32)])
    uv = _uv_proj(hp12, A1s, ab1s)
    u, v = uv[0], uv[1]

    ar = jnp.arange(B, dtype=batch1.dtype)
    starts1 = jnp.searchsorted(batch1, ar, side="left").astype(jnp.int32)
    counts1 = jnp.searchsorted(batch1, ar, side="right").astype(jnp.int32) - starts1
    starts2 = jnp.searchsorted(batch2, ar, side="left").astype(jnp.int32)
    counts2 = jnp.searchsorted(batch2, ar, side="right").astype(jnp.int32) - starts2

    g1, g2 = _attention(starts1, counts1, starts2, counts2, u, v, hp1, hp2,
                        A2.reshape(1, H), ab2.reshape(1, 1))
    return _final_mlp(g1, g2, C1, cb1, C2, cb2)


# speedup vs baseline: 2.4556x; 1.0558x over previous
import functools

import jax
import jax.numpy as jnp
from jax import lax
from jax.experimental import pallas as pl
from jax.experimental.pallas import tpu as pltpu
from jax.experimental.pallas import tpu_sc as plsc

N = 10000
F = 128
H = 128
B = 256
E = 160000
C = 86
L = 3

TILE = 48
NP = N + TILE


_CH = 128
_NCH = 80
_EPW = _NCH * _CH
_EPAD = 16 * _EPW
_RPT = 632


@functools.cache
def _build_sc_segment_sum():
    @functools.partial(
        pl.kernel,
        out_type=jax.ShapeDtypeStruct((2 * N, H), jnp.float32),
        mesh=plsc.VectorSubcoreMesh(core_axis_name="c", subcore_axis_name="s"),
        scratch_types=[
            pltpu.VMEM((_NCH // 2, _CH), jnp.int32),
            pltpu.VMEM((_NCH // 2, _CH), jnp.int32),
            pltpu.VMEM((_CH, H), jnp.float32),
            pltpu.VMEM((_CH, H), jnp.float32),
            pltpu.VMEM_SHARED((N, H), jnp.float32),
            pltpu.SemaphoreType.DMA,
            pltpu.SemaphoreType.DMA,
        ],
    )
    def sc_seg_sum(h_hbm, src_hbm, dst_hbm, zeros_hbm, out_hbm,
                   idx_s, idx_d, rows0, rows1, agg_sh, sem0, sem1):
        c = lax.axis_index("c")
        s = lax.axis_index("s")
        w = c * 16 + s
        last = N - 15 * _RPT

        @pl.when(s < 15)
        def _():
            pltpu.sync_copy(zeros_hbm, agg_sh.at[pl.ds(s * _RPT, _RPT)])

        @pl.when(s == 15)
        def _():
            pltpu.sync_copy(zeros_hbm.at[pl.ds(0, last)],
                            agg_sh.at[pl.ds(15 * _RPT, last)])

        plsc.subcore_barrier()
        half = _NCH // 2
        for p in range(2):
            pltpu.sync_copy(src_hbm.at[w, p], idx_s)
            pltpu.sync_copy(dst_hbm.at[w, p], idx_d)

            pltpu.make_async_copy(h_hbm.at[idx_s.at[0]], rows0, sem0).start()

            def body(j, carry):
                i0 = 2 * j
                pltpu.make_async_copy(h_hbm.at[idx_s.at[i0 + 1]], rows1,
                                      sem1).start()
                pltpu.make_async_copy(h_hbm.at[idx_s.at[i0]], rows0,
                                      sem0).wait()
                pltpu.sync_copy(rows0, agg_sh.at[idx_d.at[i0]], add=True)

                @pl.when(i0 + 2 < half)
                def _():
                    pltpu.make_async_copy(h_hbm.at[idx_s.at[i0 + 2]], rows0,
                                          sem0).start()

                pltpu.make_async_copy(h_hbm.at[idx_s.at[i0 + 1]], rows1,
                                      sem1).wait()
                pltpu.sync_copy(rows1, agg_sh.at[idx_d.at[i0 + 1]], add=True)
                return carry

            lax.fori_loop(0, half // 2, body, 0)

        plsc.subcore_barrier()

        @pl.when(s < 15)
        def _():
            pltpu.sync_copy(agg_sh.at[pl.ds(s * _RPT, _RPT)],
                            out_hbm.at[pl.ds(c * N + s * _RPT, _RPT)])

        @pl.when(s == 15)
        def _():
            pltpu.sync_copy(agg_sh.at[pl.ds(15 * _RPT, last)],
                            out_hbm.at[pl.ds(c * N + 15 * _RPT, last)])

    return sc_seg_sum


def _sc_segment_sum(h_cat, src, dst, zeros_blk):
    return _build_sc_segment_sum()(h_cat, src, dst, zeros_blk)


def _proj_body(x_ref, w_ref, b_ref, o_ref):
    o_ref[...] = jnp.dot(x_ref[...], w_ref[...],
                         preferred_element_type=jnp.float32) + b_ref[...]


def _input_proj(x_cat, W_in, b_in):
    blk = 2000 if (2 * N) % 2000 == 0 else 2 * N
    return pl.pallas_call(
        _proj_body,
        grid=(2 * N // blk,),
        in_specs=[
            pl.BlockSpec((blk, F), lambda i: (i, 0)),
            pl.BlockSpec((F, H), lambda i: (0, 0)),
            pl.BlockSpec((1, H), lambda i: (0, 0)),
        ],
        out_specs=pl.BlockSpec((blk, H), lambda i: (i, 0)),
        out_shape=jax.ShapeDtypeStruct((2 * N, H), jnp.float32),
    )(x_cat, W_in, b_in.reshape(1, H))


def _layer_body(agg_ref, h_ref, w_ref, b_ref, g_ref, be_ref, o_ref):
    agg = agg_ref[0]
    z = jnp.dot(agg, w_ref[...], preferred_element_type=jnp.float32) + b_ref[...]
    mu = jnp.mean(z, axis=0, keepdims=True)
    var = jnp.mean((z - mu) ** 2, axis=0, keepdims=True)
    hn = g_ref[...] * (z - mu) / jnp.sqrt(var + 1e-5) + be_ref[...]
    o_ref[...] = h_ref[...] + jnp.maximum(hn, 0.0)


def _layer_update(agg, h_cat, Wc_i, bc_i, gamma_i, beta_i):
    return pl.pallas_call(
        _layer_body,
        grid=(2,),
        in_specs=[
            pl.BlockSpec((1, N, H), lambda g: (g, 0, 0)),
            pl.BlockSpec((N, H), lambda g: (g, 0)),
            pl.BlockSpec((H, H), lambda g: (0, 0)),
            pl.BlockSpec((1, H), lambda g: (0, 0)),
            pl.BlockSpec((1, H), lambda g: (0, 0)),
            pl.BlockSpec((1, H), lambda g: (0, 0)),
        ],
        out_specs=pl.BlockSpec((N, H), lambda g: (g, 0)),
        out_shape=jax.ShapeDtypeStruct((2 * N, H), jnp.float32),
    )(agg.reshape(2, N, H), h_cat, Wc_i, bc_i.reshape(1, H),
      gamma_i.reshape(1, H), beta_i.reshape(1, H))


def _uv_body(h_ref, a_ref, b_ref, o_ref):
    o_ref[0] = jnp.dot(h_ref[0], a_ref[0],
                       preferred_element_type=jnp.float32) + b_ref[0]


def _uv_proj(hp12, A1s, ab1s):
    return pl.pallas_call(
        _uv_body,
        grid=(2,),
        in_specs=[
            pl.BlockSpec((1, NP, H), lambda g: (g, 0, 0)),
            pl.BlockSpec((1, H, H), lambda g: (g, 0, 0)),
            pl.BlockSpec((1, 1, H), lambda g: (g, 0, 0)),
        ],
        out_specs=pl.BlockSpec((1, NP, H), lambda g: (g, 0, 0)),
        out_shape=jax.ShapeDtypeStruct((2, NP, H), jnp.float32),
    )(hp12, A1s, ab1s)


def _attn_body(s1_ref, c1_ref, s2_ref, c2_ref,
               u_ref, v_ref, h1_ref, h2_ref, a2_ref, ab2_ref,
               g1_ref, g2_ref, att1_buf, att2_buf):
    b = pl.program_id(0)
    s1 = s1_ref[b]
    n1 = c1_ref[b]
    s2 = s2_ref[b]
    n2 = c2_ref[b]
    nr1 = (n1 + TILE - 1) // TILE
    nc2 = (n2 + TILE - 1) // TILE

    a2 = a2_ref[...]
    ab2 = ab2_ref[0, 0]
    smax = jnp.sum(jnp.abs(a2)) + jnp.abs(ab2)

    def zbody(ct, carry):
        att2_buf[pl.ds(ct * TILE, TILE), :] = jnp.zeros((TILE, 1), jnp.float32)
        return carry
    lax.fori_loop(0, nc2, zbody, 0)

    def rbody(rt, denom):
        u_t = u_ref[pl.ds(s1 + rt * TILE, TILE), :]
        rrem = n1 - rt * TILE

        def cbody(ct, carry):
            att1_acc, dn = carry
            v_t = v_ref[pl.ds(s2 + ct * TILE, TILE), :]
            crem = n2 - ct * TILE
            t3 = jnp.tanh(u_t[:, None, :] + v_t[None, :, :])
            S = jnp.sum(t3 * a2[None, :, :], axis=-1) + ab2
            rmask = lax.broadcasted_iota(jnp.int32, (TILE, TILE), 0) < rrem
            cmask = lax.broadcasted_iota(jnp.int32, (TILE, TILE), 1) < crem
            e = jnp.where(rmask & cmask, jnp.exp(S - smax), 0.0)
            att1_acc = att1_acc + jnp.sum(e, axis=1, keepdims=True)
            col = jnp.sum(e.T, axis=1, keepdims=True)
            att2_buf[pl.ds(ct * TILE, TILE), :] = (
                att2_buf[pl.ds(ct * TILE, TILE), :] + col)
            return att1_acc, dn + jnp.sum(e)

        att1_acc, denom = lax.fori_loop(
            0, nc2, cbody, (jnp.zeros((TILE, 1), jnp.float32), denom))
        att1_buf[pl.ds(rt * TILE, TILE), :] = att1_acc
        return denom

    denom = lax.fori_loop(0, nr1, rbody, jnp.float32(0.0))
    dsafe = jnp.where(denom > 0.0, denom, 1.0)

    def g1body(rt, acc):
        w = att1_buf[pl.ds(rt * TILE, TILE), :] / dsafe
        h_t = h1_ref[pl.ds(s1 + rt * TILE, TILE), :]
        return acc + jnp.sum(w * h_t, axis=0, keepdims=True)

    g1_ref[0] = lax.fori_loop(0, nr1, g1body, jnp.zeros((1, H), jnp.float32))

    def g2body(ct, acc):
        w = att2_buf[pl.ds(ct * TILE, TILE), :] / dsafe
        h_t = h2_ref[pl.ds(s2 + ct * TILE, TILE), :]
        return acc + jnp.sum(w * h_t, axis=0, keepdims=True)

    g2_ref[0] = lax.fori_loop(0, nc2, g2body, jnp.zeros((1, H), jnp.float32))


def _attention(starts1, counts1, starts2, counts2, u, v, h1p, h2p, a2t, ab2m):
    grid_spec = pltpu.PrefetchScalarGridSpec(
        num_scalar_prefetch=4,
        grid=(B,),
        in_specs=[
            pl.BlockSpec((NP, H), lambda b, *_: (0, 0)),
            pl.BlockSpec((NP, H), lambda b, *_: (0, 0)),
            pl.BlockSpec((NP, H), lambda b, *_: (0, 0)),
            pl.BlockSpec((NP, H), lambda b, *_: (0, 0)),
            pl.BlockSpec((1, H), lambda b, *_: (0, 0)),
            pl.BlockSpec((1, 1), lambda b, *_: (0, 0)),
        ],
        out_specs=[
            pl.BlockSpec((1, 1, H), lambda b, *_: (b, 0, 0)),
            pl.BlockSpec((1, 1, H), lambda b, *_: (b, 0, 0)),
        ],
        scratch_shapes=[
            pltpu.VMEM((NP, 1), jnp.float32),
            pltpu.VMEM((NP, 1), jnp.float32),
        ],
    )
    g1, g2 = pl.pallas_call(
        _attn_body,
        grid_spec=grid_spec,
        out_shape=[
            jax.ShapeDtypeStruct((B, 1, H), jnp.float32),
            jax.ShapeDtypeStruct((B, 1, H), jnp.float32),
        ],
    )(starts1, counts1, starts2, counts2, u, v, h1p, h2p, a2t, ab2m)
    return g1.reshape(B, H), g2.reshape(B, H)


def _mlp_body(g1_ref, g2_ref, c1a_ref, c1b_ref, cb1_ref, c2_ref, cb2_ref, o_ref):
    hp = jnp.dot(g1_ref[...], c1a_ref[...], preferred_element_type=jnp.float32)
    hp = hp + jnp.dot(g2_ref[...], c1b_ref[...], preferred_element_type=jnp.float32)
    hp = jnp.maximum(hp + cb1_ref[...], 0.0)
    o_ref[...] = jnp.dot(hp, c2_ref[...],
                         preferred_element_type=jnp.float32) + cb2_ref[...]


def _final_mlp(g1, g2, C1, cb1, C2, cb2):
    return pl.pallas_call(
        _mlp_body,
        out_shape=jax.ShapeDtypeStruct((B, C), jnp.float32),
    )(g1, g2, C1[:H], C1[H:], cb1.reshape(1, H), C2, cb2.reshape(1, C))


def kernel(x1, x2, edge_index1, edge_index2, batch1, batch2, W_in, b_in, Wc, bc,
           gamma, beta, A1, ab1, A2, ab2, C1, cb1, C2, cb2):
    x_cat = jnp.concatenate([x1, x2], axis=0)
    pad = _EPAD - E
    spad = 2 * N + jnp.arange(pad, dtype=jnp.int32) % 128
    dpad = jnp.arange(pad, dtype=jnp.int32) % N
    src1 = jnp.concatenate([edge_index1[0].astype(jnp.int32), spad])
    src2 = jnp.concatenate([edge_index2[0].astype(jnp.int32) + N, spad])
    dst1 = jnp.concatenate([edge_index1[1].astype(jnp.int32), dpad])
    dst2 = jnp.concatenate([edge_index2[1].astype(jnp.int32), dpad])
    src = jnp.stack([src1, src2]).reshape(32, 2, _NCH // 2, _CH)
    dst = jnp.stack([dst1, dst2]).reshape(32, 2, _NCH // 2, _CH)
    zeros_blk = jnp.zeros((_RPT, H), jnp.float32)
    zrow = jnp.zeros((128, H), jnp.float32)

    h_cat = _input_proj(x_cat, W_in, b_in)
    for i in range(L):
        agg = _sc_segment_sum(jnp.concatenate([h_cat, zrow]), src, dst,
                              zeros_blk)
        h_cat = _layer_update(agg, h_cat, Wc[i], bc[i], gamma[i], beta[i])

    h1 = h_cat[:N]
    h2 = h_cat[N:]
    hp1 = jnp.zeros((NP, H), jnp.float32).at[:N].set(h1)
    hp2 = jnp.zeros((NP, H), jnp.float32).at[:N].set(h2)
    hp12 = jnp.stack([hp1, hp2])
    A1s = A1.reshape(2, H, H)
    ab1s = jnp.stack([ab1.reshape(1, H), jnp.zeros((1, H), jnp.float32)])
    uv = _uv_proj(hp12, A1s, ab1s)
    u, v = uv[0], uv[1]

    ar = jnp.arange(B, dtype=batch1.dtype)
    starts1 = jnp.searchsorted(batch1, ar, side="left").astype(jnp.int32)
    counts1 = jnp.searchsorted(batch1, ar, side="right").astype(jnp.int32) - starts1
    starts2 = jnp.searchsorted(batch2, ar, side="left").astype(jnp.int32)
    counts2 = jnp.searchsorted(batch2, ar, side="right").astype(jnp.int32) - starts2

    g1, g2 = _attention(starts1, counts1, starts2, counts2, u, v, hp1, hp2,
                        A2.reshape(1, H), ab2.reshape(1, 1))
    return _final_mlp(g1, g2, C1, cb1, C2, cb2)


# speedup vs baseline: 2.6353x; 1.0732x over previous
import functools

import jax
import jax.numpy as jnp
from jax import lax
from jax.experimental import pallas as pl
from jax.experimental.pallas import tpu as pltpu
from jax.experimental.pallas import tpu_sc as plsc

N = 10000
F = 128
H = 128
B = 256
E = 160000
C = 86
L = 3

TILE = 48
NP = N + TILE


_CH = 128
_NCH = 80
_EPW = _NCH * _CH
_EPAD = 16 * _EPW
_RPT = 632


@functools.cache
def _build_sc_segment_sum():
    @functools.partial(
        pl.kernel,
        out_type=jax.ShapeDtypeStruct((2 * N, H), jnp.float32),
        mesh=plsc.VectorSubcoreMesh(core_axis_name="c", subcore_axis_name="s"),
        scratch_types=[
            pltpu.VMEM((_NCH // 2, _CH), jnp.int32),
            pltpu.VMEM((_NCH // 2, _CH), jnp.int32),
            pltpu.VMEM((_CH, H), jnp.float32),
            pltpu.VMEM((_CH, H), jnp.float32),
            pltpu.VMEM_SHARED((N, H), jnp.float32),
            pltpu.SemaphoreType.DMA,
            pltpu.SemaphoreType.DMA,
        ],
    )
    def sc_seg_sum(h_hbm, src_hbm, dst_hbm, zeros_hbm, out_hbm,
                   idx_s, idx_d, rows0, rows1, agg_sh, sem0, sem1):
        c = lax.axis_index("c")
        s = lax.axis_index("s")
        w = c * 16 + s
        last = N - 15 * _RPT

        @pl.when(s < 15)
        def _():
            pltpu.sync_copy(zeros_hbm, agg_sh.at[pl.ds(s * _RPT, _RPT)])

        @pl.when(s == 15)
        def _():
            pltpu.sync_copy(zeros_hbm.at[pl.ds(0, last)],
                            agg_sh.at[pl.ds(15 * _RPT, last)])

        plsc.subcore_barrier()
        half = _NCH // 2
        for p in range(2):
            pltpu.sync_copy(src_hbm.at[w, p], idx_s)
            pltpu.sync_copy(dst_hbm.at[w, p], idx_d)

            pltpu.make_async_copy(h_hbm.at[idx_s.at[0]], rows0, sem0).start()

            def body(j, carry):
                i0 = 2 * j
                pltpu.make_async_copy(h_hbm.at[idx_s.at[i0 + 1]], rows1,
                                      sem1).start()
                pltpu.make_async_copy(h_hbm.at[idx_s.at[i0]], rows0,
                                      sem0).wait()
                pltpu.sync_copy(rows0, agg_sh.at[idx_d.at[i0]], add=True)

                @pl.when(i0 + 2 < half)
                def _():
                    pltpu.make_async_copy(h_hbm.at[idx_s.at[i0 + 2]], rows0,
                                          sem0).start()

                pltpu.make_async_copy(h_hbm.at[idx_s.at[i0 + 1]], rows1,
                                      sem1).wait()
                pltpu.sync_copy(rows1, agg_sh.at[idx_d.at[i0 + 1]], add=True)
                return carry

            lax.fori_loop(0, half // 2, body, 0)

        plsc.subcore_barrier()

        @pl.when(s < 15)
        def _():
            pltpu.sync_copy(agg_sh.at[pl.ds(s * _RPT, _RPT)],
                            out_hbm.at[pl.ds(c * N + s * _RPT, _RPT)])

        @pl.when(s == 15)
        def _():
            pltpu.sync_copy(agg_sh.at[pl.ds(15 * _RPT, last)],
                            out_hbm.at[pl.ds(c * N + 15 * _RPT, last)])

    return sc_seg_sum


def _sc_segment_sum(h_cat, src, dst, zeros_blk):
    return _build_sc_segment_sum()(h_cat, src, dst, zeros_blk)


def _proj_body(x_ref, w_ref, b_ref, o_ref):
    o_ref[0, :N] = jnp.dot(x_ref[0], w_ref[...],
                           preferred_element_type=jnp.float32) + b_ref[...]
    o_ref[0, N:] = jnp.zeros((NP - N, H), jnp.float32)


def _input_proj(x12, W_in, b_in):
    return pl.pallas_call(
        _proj_body,
        grid=(2,),
        in_specs=[
            pl.BlockSpec((1, N, F), lambda g: (g, 0, 0)),
            pl.BlockSpec((F, H), lambda g: (0, 0)),
            pl.BlockSpec((1, H), lambda g: (0, 0)),
        ],
        out_specs=pl.BlockSpec((1, NP, H), lambda g: (g, 0, 0)),
        out_shape=jax.ShapeDtypeStruct((2, NP, H), jnp.float32),
    )(x12, W_in, b_in.reshape(1, H))


def _layer_body(agg_ref, h_ref, w_ref, b_ref, g_ref, be_ref, o_ref):
    agg = agg_ref[0]
    z = jnp.dot(agg, w_ref[...], preferred_element_type=jnp.float32) + b_ref[...]
    mu = jnp.mean(z, axis=0, keepdims=True)
    var = jnp.mean((z - mu) ** 2, axis=0, keepdims=True)
    hn = g_ref[...] * (z - mu) / jnp.sqrt(var + 1e-5) + be_ref[...]
    o_ref[0, :N] = h_ref[0, :N] + jnp.maximum(hn, 0.0)
    o_ref[0, N:] = jnp.zeros((NP - N, H), jnp.float32)


def _layer_update(agg, h, Wc_i, bc_i, gamma_i, beta_i):
    return pl.pallas_call(
        _layer_body,
        grid=(2,),
        in_specs=[
            pl.BlockSpec((1, N, H), lambda g: (g, 0, 0)),
            pl.BlockSpec((1, NP, H), lambda g: (g, 0, 0)),
            pl.BlockSpec((H, H), lambda g: (0, 0)),
            pl.BlockSpec((1, H), lambda g: (0, 0)),
            pl.BlockSpec((1, H), lambda g: (0, 0)),
            pl.BlockSpec((1, H), lambda g: (0, 0)),
        ],
        out_specs=pl.BlockSpec((1, NP, H), lambda g: (g, 0, 0)),
        out_shape=jax.ShapeDtypeStruct((2, NP, H), jnp.float32),
    )(agg.reshape(2, N, H), h, Wc_i, bc_i.reshape(1, H),
      gamma_i.reshape(1, H), beta_i.reshape(1, H))


def _uv_body(h_ref, a_ref, b_ref, o_ref):
    o_ref[0] = jnp.dot(h_ref[0], a_ref[0],
                       preferred_element_type=jnp.float32) + b_ref[0]


def _uv_proj(hp12, A1s, ab1s):
    return pl.pallas_call(
        _uv_body,
        grid=(2,),
        in_specs=[
            pl.BlockSpec((1, NP, H), lambda g: (g, 0, 0)),
            pl.BlockSpec((1, H, H), lambda g: (g, 0, 0)),
            pl.BlockSpec((1, 1, H), lambda g: (g, 0, 0)),
        ],
        out_specs=pl.BlockSpec((1, NP, H), lambda g: (g, 0, 0)),
        out_shape=jax.ShapeDtypeStruct((2, NP, H), jnp.float32),
    )(hp12, A1s, ab1s)


def _attn_body(s1_ref, c1_ref, s2_ref, c2_ref,
               uv_ref, h_ref, a2_ref, ab2_ref,
               g1_ref, g2_ref, att1_buf, att2_buf):
    b = pl.program_id(0)
    s1 = s1_ref[b]
    n1 = c1_ref[b]
    s2 = s2_ref[b]
    n2 = c2_ref[b]
    nr1 = (n1 + TILE - 1) // TILE
    nc2 = (n2 + TILE - 1) // TILE

    a2 = a2_ref[...]
    ab2 = ab2_ref[0, 0]
    smax = jnp.sum(jnp.abs(a2)) + jnp.abs(ab2)

    def zbody(ct, carry):
        att2_buf[pl.ds(ct * TILE, TILE), :] = jnp.zeros((TILE, 1), jnp.float32)
        return carry
    lax.fori_loop(0, nc2, zbody, 0)

    def rbody(rt, denom):
        u_t = uv_ref[0, pl.ds(s1 + rt * TILE, TILE), :]
        rrem = n1 - rt * TILE

        def cbody(ct, carry):
            att1_acc, dn = carry
            v_t = uv_ref[1, pl.ds(s2 + ct * TILE, TILE), :]
            crem = n2 - ct * TILE
            t3 = jnp.tanh(u_t[:, None, :] + v_t[None, :, :])
            S = jnp.sum(t3 * a2[None, :, :], axis=-1) + ab2
            rmask = lax.broadcasted_iota(jnp.int32, (TILE, TILE), 0) < rrem
            cmask = lax.broadcasted_iota(jnp.int32, (TILE, TILE), 1) < crem
            e = jnp.where(rmask & cmask, jnp.exp(S - smax), 0.0)
            att1_acc = att1_acc + jnp.sum(e, axis=1, keepdims=True)
            col = jnp.sum(e.T, axis=1, keepdims=True)
            att2_buf[pl.ds(ct * TILE, TILE), :] = (
                att2_buf[pl.ds(ct * TILE, TILE), :] + col)
            return att1_acc, dn + jnp.sum(e)

        att1_acc, denom = lax.fori_loop(
            0, nc2, cbody, (jnp.zeros((TILE, 1), jnp.float32), denom))
        att1_buf[pl.ds(rt * TILE, TILE), :] = att1_acc
        return denom

    denom = lax.fori_loop(0, nr1, rbody, jnp.float32(0.0))
    dsafe = jnp.where(denom > 0.0, denom, 1.0)

    def g1body(rt, acc):
        w = att1_buf[pl.ds(rt * TILE, TILE), :] / dsafe
        h_t = h_ref[0, pl.ds(s1 + rt * TILE, TILE), :]
        return acc + jnp.sum(w * h_t, axis=0, keepdims=True)

    g1_ref[0] = lax.fori_loop(0, nr1, g1body, jnp.zeros((1, H), jnp.float32))

    def g2body(ct, acc):
        w = att2_buf[pl.ds(ct * TILE, TILE), :] / dsafe
        h_t = h_ref[1, pl.ds(s2 + ct * TILE, TILE), :]
        return acc + jnp.sum(w * h_t, axis=0, keepdims=True)

    g2_ref[0] = lax.fori_loop(0, nc2, g2body, jnp.zeros((1, H), jnp.float32))


def _attention(starts1, counts1, starts2, counts2, uv, h, a2t, ab2m):
    grid_spec = pltpu.PrefetchScalarGridSpec(
        num_scalar_prefetch=4,
        grid=(B,),
        in_specs=[
            pl.BlockSpec((2, NP, H), lambda b, *_: (0, 0, 0)),
            pl.BlockSpec((2, NP, H), lambda b, *_: (0, 0, 0)),
            pl.BlockSpec((1, H), lambda b, *_: (0, 0)),
            pl.BlockSpec((1, 1), lambda b, *_: (0, 0)),
        ],
        out_specs=[
            pl.BlockSpec((1, 1, H), lambda b, *_: (b, 0, 0)),
            pl.BlockSpec((1, 1, H), lambda b, *_: (b, 0, 0)),
        ],
        scratch_shapes=[
            pltpu.VMEM((NP, 1), jnp.float32),
            pltpu.VMEM((NP, 1), jnp.float32),
        ],
    )
    g1, g2 = pl.pallas_call(
        _attn_body,
        grid_spec=grid_spec,
        out_shape=[
            jax.ShapeDtypeStruct((B, 1, H), jnp.float32),
            jax.ShapeDtypeStruct((B, 1, H), jnp.float32),
        ],
    )(starts1, counts1, starts2, counts2, uv, h, a2t, ab2m)
    return g1.reshape(B, H), g2.reshape(B, H)


def _mlp_body(g1_ref, g2_ref, c1a_ref, c1b_ref, cb1_ref, c2_ref, cb2_ref, o_ref):
    hp = jnp.dot(g1_ref[...], c1a_ref[...], preferred_element_type=jnp.float32)
    hp = hp + jnp.dot(g2_ref[...], c1b_ref[...], preferred_element_type=jnp.float32)
    hp = jnp.maximum(hp + cb1_ref[...], 0.0)
    o_ref[...] = jnp.dot(hp, c2_ref[...],
                         preferred_element_type=jnp.float32) + cb2_ref[...]


def _final_mlp(g1, g2, C1, cb1, C2, cb2):
    return pl.pallas_call(
        _mlp_body,
        out_shape=jax.ShapeDtypeStruct((B, C), jnp.float32),
    )(g1, g2, C1[:H], C1[H:], cb1.reshape(1, H), C2, cb2.reshape(1, C))


def kernel(x1, x2, edge_index1, edge_index2, batch1, batch2, W_in, b_in, Wc, bc,
           gamma, beta, A1, ab1, A2, ab2, C1, cb1, C2, cb2):
    x12 = jnp.stack([x1, x2])
    pad = _EPAD - E
    k = jnp.arange(pad, dtype=jnp.int32) % 96
    spad = jnp.where(k < 48, N + k, NP + N + k - 48)
    dpad = jnp.arange(pad, dtype=jnp.int32) % N
    src1 = jnp.concatenate([edge_index1[0].astype(jnp.int32), spad])
    src2 = jnp.concatenate([edge_index2[0].astype(jnp.int32) + NP, spad])
    dst1 = jnp.concatenate([edge_index1[1].astype(jnp.int32), dpad])
    dst2 = jnp.concatenate([edge_index2[1].astype(jnp.int32), dpad])
    src = jnp.stack([src1, src2]).reshape(32, 2, _NCH // 2, _CH)
    dst = jnp.stack([dst1, dst2]).reshape(32, 2, _NCH // 2, _CH)
    zeros_blk = jnp.zeros((_RPT, H), jnp.float32)

    h = _input_proj(x12, W_in, b_in)
    for i in range(L):
        agg = _sc_segment_sum(h.reshape(2 * NP, H), src, dst, zeros_blk)
        h = _layer_update(agg, h, Wc[i], bc[i], gamma[i], beta[i])

    A1s = A1.reshape(2, H, H)
    ab1s = jnp.stack([ab1.reshape(1, H), jnp.zeros((1, H), jnp.float32)])
    uv = _uv_proj(h, A1s, ab1s)

    ar = jnp.arange(B, dtype=batch1.dtype)
    starts1 = jnp.searchsorted(batch1, ar, side="left").astype(jnp.int32)
    counts1 = jnp.searchsorted(batch1, ar, side="right").astype(jnp.int32) - starts1
    starts2 = jnp.searchsorted(batch2, ar, side="left").astype(jnp.int32)
    counts2 = jnp.searchsorted(batch2, ar, side="right").astype(jnp.int32) - starts2

    g1, g2 = _attention(starts1, counts1, starts2, counts2, uv, h,
                        A2.reshape(1, H), ab2.reshape(1, 1))
    return _final_mlp(g1, g2, C1, cb1, C2, cb2)
